# baseline (speedup 1.0000x reference)
import jax
import jax.numpy as jnp
from jax import lax
from jax.experimental import pallas as pl
from jax.experimental.pallas import tpu as pltpu

B, D, T = 32, 64, 576
K = 1024
N = B * T
EPS = 1e-12
BB = 8
W = BB * T
C = D + 8
INV_ND = 1.0 / float(N * D)


def _vq_kernel(x_ref, emb_ref, embu_ref, quant_ref, idx_ref, loss_ref, cat_ref):
    g = pl.program_id(0)

    @pl.when(g == 0)
    def _init():
        loss_ref[...] = jnp.zeros((1, 1), jnp.float32)
        cat_ref[:, :D] = emb_ref[...].astype(jnp.bfloat16)
        embu = embu_ref[...]
        un = jnp.sqrt(jnp.sum(embu * embu, axis=1, keepdims=True))
        u_hi = un.astype(jnp.bfloat16)
        u_lo = (un - u_hi.astype(jnp.float32)).astype(jnp.bfloat16)
        k_col = lax.broadcasted_iota(jnp.int32, (K, 1), 0)
        cat_ref[:, D:D + 1] = (k_col >> 5).astype(jnp.bfloat16)
        cat_ref[:, D + 1:D + 2] = (k_col & 31).astype(jnp.bfloat16)
        cat_ref[:, D + 2:D + 3] = jnp.ones((K, 1), jnp.bfloat16)
        cat_ref[:, D + 3:D + 4] = u_hi
        cat_ref[:, D + 4:D + 5] = u_lo
        cat_ref[:, D + 5:] = jnp.zeros((K, 3), jnp.bfloat16)

    x = jnp.concatenate([x_ref[i] for i in range(BB)], axis=1)
    emb = emb_ref[...]

    nrm2 = jnp.sum(x * x, axis=0, keepdims=True)
    nrm = jnp.sqrt(nrm2)
    xn = x / jnp.maximum(nrm, EPS)

    dist = lax.dot_general(
        emb, xn, (((1,), (0,)), ((), ())),
        preferred_element_type=jnp.float32,
    )

    maxval = jnp.max(dist, axis=0, keepdims=True)
    mask = (dist >= maxval).astype(jnp.bfloat16)

    combo = lax.dot_general(
        cat_ref[...], mask, (((0,), (0,)), ((), ())),
        preferred_element_type=jnp.float32,
    )
    idxf = combo[D] * 32.0 + combo[D + 1]
    cnt = combo[D + 2]
    unorm = combo[D + 3] + combo[D + 4]

    idx = idxf.astype(jnp.int32)
    for i in range(BB):
        idx_ref[i, 0] = idx[i * T:(i + 1) * T]
        quant_ref[i] = combo[:D, i * T:(i + 1) * T]
    part = jnp.sum(nrm2[0] - 2.0 * maxval[0] * nrm[0] * unorm + unorm * unorm)
    loss_ref[...] += (part * INV_ND).reshape(1, 1)

    tie = jnp.max(cnt) > 1.5

    @pl.when(tie)
    def _exact():
        iota_f = lax.broadcasted_iota(jnp.int32, (K, W), 0).astype(jnp.float32)
        idxe = jnp.min(jnp.where(dist >= maxval, iota_f, float(K)), axis=0)
        idxi = idxe.astype(jnp.int32)
        for i in range(BB):
            idx_ref[i, 0] = idxi[i * T:(i + 1) * T]


def kernel(inputs, embedding, embedding_unnormalized):
    quant, idx3, loss_sum = pl.pallas_call(
        _vq_kernel,
        grid=(B // BB,),
        in_specs=[
            pl.BlockSpec((BB, D, T), lambda g: (g, 0, 0)),
            pl.BlockSpec((K, D), lambda g: (0, 0)),
            pl.BlockSpec((K, D), lambda g: (0, 0)),
        ],
        out_specs=[
            pl.BlockSpec((BB, D, T), lambda g: (g, 0, 0)),
            pl.BlockSpec((BB, 1, T), lambda g: (g, 0, 0)),
            pl.BlockSpec((1, 1), lambda g: (0, 0)),
        ],
        out_shape=[
            jax.ShapeDtypeStruct((B, D, T), jnp.float32),
            jax.ShapeDtypeStruct((B, 1, T), jnp.int32),
            jax.ShapeDtypeStruct((1, 1), jnp.float32),
        ],
        scratch_shapes=[pltpu.VMEM((K, C), jnp.bfloat16)],
    )(inputs, embedding, embedding_unnormalized)

    loss = loss_sum.reshape(())
    encoding_indices = idx3.reshape(N)
    return (quant, loss, loss, encoding_indices)


# speedup vs baseline: 1.0092x; 1.0092x over previous
import jax
import jax.numpy as jnp
from jax import lax
from jax.experimental import pallas as pl
from jax.experimental.pallas import tpu as pltpu

B, D, T = 32, 64, 576
K = 1024
N = B * T
EPS = 1e-12
BB = 8
W = BB * T
C = D + 8
INV_ND = 1.0 / float(N * D)


def _vq_kernel(x_ref, emb_ref, aux_ref, quant_ref, idx_ref, loss_ref, cat_ref):
    g = pl.program_id(0)

    @pl.when(g == 0)
    def _init():
        loss_ref[...] = jnp.zeros((1, 1), jnp.float32)
        cat_ref[:, :D] = emb_ref[...].astype(jnp.bfloat16)
        cat_ref[:, D:] = aux_ref[...]

    x = jnp.concatenate([x_ref[i] for i in range(BB)], axis=1)
    emb = emb_ref[...]

    nrm2 = jnp.sum(x * x, axis=0, keepdims=True)
    nrm = jnp.sqrt(nrm2)
    xn = x / jnp.maximum(nrm, EPS)

    H = K // 2
    dist1 = lax.dot_general(
        emb[:H], xn, (((1,), (0,)), ((), ())),
        preferred_element_type=jnp.float32,
    )
    m1 = jnp.max(dist1, axis=0, keepdims=True)
    dist2 = lax.dot_general(
        emb[H:], xn, (((1,), (0,)), ((), ())),
        preferred_element_type=jnp.float32,
    )
    m2 = jnp.max(dist2, axis=0, keepdims=True)
    maxval = jnp.maximum(m1, m2)
    mask1 = (dist1 >= maxval).astype(jnp.bfloat16)
    mask2 = (dist2 >= maxval).astype(jnp.bfloat16)

    combo = lax.dot_general(
        cat_ref[:H], mask1, (((0,), (0,)), ((), ())),
        preferred_element_type=jnp.float32,
    ) + lax.dot_general(
        cat_ref[H:], mask2, (((0,), (0,)), ((), ())),
        preferred_element_type=jnp.float32,
    )
    idxf = combo[D] * 32.0 + combo[D + 1]
    cnt = combo[D + 2]
    unorm = combo[D + 3] + combo[D + 4]

    idx = idxf.astype(jnp.int32)
    for i in range(BB):
        idx_ref[i, 0] = idx[i * T:(i + 1) * T]
        quant_ref[i] = combo[:D, i * T:(i + 1) * T]
    part = jnp.sum(nrm2[0] - 2.0 * maxval[0] * nrm[0] * unorm + unorm * unorm)
    loss_ref[...] += (part * INV_ND).reshape(1, 1)

    tie = jnp.max(cnt) > 1.5

    @pl.when(tie)
    def _exact():
        iota_f = lax.broadcasted_iota(jnp.int32, (H, W), 0).astype(jnp.float32)
        e1 = jnp.min(jnp.where(dist1 >= maxval, iota_f, float(K)), axis=0)
        e2 = jnp.min(jnp.where(dist2 >= maxval, iota_f + float(H), float(K)),
                     axis=0)
        idxe = jnp.minimum(e1, e2)
        idxi = idxe.astype(jnp.int32)
        for i in range(BB):
            idx_ref[i, 0] = idxi[i * T:(i + 1) * T]


def kernel(inputs, embedding, embedding_unnormalized):
    k_iota = jnp.arange(K, dtype=jnp.int32)
    unorm = jnp.linalg.norm(embedding_unnormalized, axis=1)
    u_hi = unorm.astype(jnp.bfloat16)
    u_lo = (unorm - u_hi.astype(jnp.float32)).astype(jnp.bfloat16)
    aux = jnp.stack(
        [(k_iota >> 5).astype(jnp.bfloat16),
         (k_iota & 31).astype(jnp.bfloat16),
         jnp.ones((K,), jnp.bfloat16),
         u_hi, u_lo]
        + [jnp.zeros((K,), jnp.bfloat16)] * 3,
        axis=1,
    )

    quant, idx3, loss_sum = pl.pallas_call(
        _vq_kernel,
        grid=(B // BB,),
        in_specs=[
            pl.BlockSpec((BB, D, T), lambda g: (g, 0, 0)),
            pl.BlockSpec((K, D), lambda g: (0, 0)),
            pl.BlockSpec((K, 8), lambda g: (0, 0)),
        ],
        out_specs=[
            pl.BlockSpec((BB, D, T), lambda g: (g, 0, 0)),
            pl.BlockSpec((BB, 1, T), lambda g: (g, 0, 0)),
            pl.BlockSpec((1, 1), lambda g: (0, 0)),
        ],
        out_shape=[
            jax.ShapeDtypeStruct((B, D, T), jnp.float32),
            jax.ShapeDtypeStruct((B, 1, T), jnp.int32),
            jax.ShapeDtypeStruct((1, 1), jnp.float32),
        ],
        scratch_shapes=[pltpu.VMEM((K, C), jnp.bfloat16)],
    )(inputs, embedding, aux)

    loss = loss_sum.reshape(())
    encoding_indices = idx3.reshape(N)
    return (quant, loss, loss, encoding_indices)


# speedup vs baseline: 1.0134x; 1.0041x over previous
import jax
import jax.numpy as jnp
from jax import lax
from jax.experimental import pallas as pl
from jax.experimental.pallas import tpu as pltpu

B, D, T = 32, 64, 576
K = 1024
N = B * T
EPS = 1e-12
BB = 8
W = BB * T
C = D + 8
INV_ND = 1.0 / float(N * D)


def _vq_kernel(x_ref, emb_ref, aux_ref, quant_ref, idx_ref, loss_ref, cat_ref):
    g = pl.program_id(0)

    @pl.when(g == 0)
    def _init():
        loss_ref[...] = jnp.zeros((1, 1), jnp.float32)
        cat_ref[:, :D] = emb_ref[...].astype(jnp.bfloat16)
        cat_ref[:, D:] = aux_ref[...]

    x = jnp.concatenate([x_ref[i] for i in range(BB)], axis=1)
    emb = emb_ref[...]

    nrm2 = jnp.sum(x * x, axis=0, keepdims=True)
    nrm = jnp.sqrt(nrm2)
    xn = x / jnp.maximum(nrm, EPS)

    dist = lax.dot_general(
        emb, xn, (((1,), (0,)), ((), ())),
        preferred_element_type=jnp.float32,
    )

    maxval = jnp.max(dist, axis=0, keepdims=True)
    mask = (dist >= maxval).astype(jnp.bfloat16)

    combo = lax.dot_general(
        cat_ref[...], mask, (((0,), (0,)), ((), ())),
        preferred_element_type=jnp.float32,
    )
    idxf = combo[D] * 32.0 + combo[D + 1]
    cnt = combo[D + 2]
    unorm = combo[D + 3] + combo[D + 4]

    idx = idxf.astype(jnp.int32)
    for i in range(BB):
        idx_ref[i, 0] = idx[i * T:(i + 1) * T]
        quant_ref[i] = combo[:D, i * T:(i + 1) * T]
    part = jnp.sum(nrm2[0] - 2.0 * maxval[0] * nrm[0] * unorm + unorm * unorm)
    loss_ref[...] += (part * INV_ND).reshape(1, 1)

    tie = jnp.max(cnt) > 1.5

    @pl.when(tie)
    def _exact():
        iota_f = lax.broadcasted_iota(jnp.int32, (K, W), 0).astype(jnp.float32)
        idxe = jnp.min(jnp.where(dist >= maxval, iota_f, float(K)), axis=0)
        idxi = idxe.astype(jnp.int32)
        for i in range(BB):
            idx_ref[i, 0] = idxi[i * T:(i + 1) * T]


def kernel(inputs, embedding, embedding_unnormalized):
    k_iota = jnp.arange(K, dtype=jnp.int32)
    unorm = jnp.linalg.norm(embedding_unnormalized, axis=1)
    u_hi = unorm.astype(jnp.bfloat16)
    u_lo = (unorm - u_hi.astype(jnp.float32)).astype(jnp.bfloat16)
    aux = jnp.stack(
        [(k_iota >> 5).astype(jnp.bfloat16),
         (k_iota & 31).astype(jnp.bfloat16),
         jnp.ones((K,), jnp.bfloat16),
         u_hi, u_lo]
        + [jnp.zeros((K,), jnp.bfloat16)] * 3,
        axis=1,
    )

    quant, idx3, loss_sum = pl.pallas_call(
        _vq_kernel,
        grid=(B // BB,),
        in_specs=[
            pl.BlockSpec((BB, D, T), lambda g: (g, 0, 0)),
            pl.BlockSpec((K, D), lambda g: (0, 0)),
            pl.BlockSpec((K, 8), lambda g: (0, 0)),
        ],
        out_specs=[
            pl.BlockSpec((BB, D, T), lambda g: (g, 0, 0)),
            pl.BlockSpec((BB, 1, T), lambda g: (g, 0, 0)),
            pl.BlockSpec((1, 1), lambda g: (0, 0)),
        ],
        out_shape=[
            jax.ShapeDtypeStruct((B, D, T), jnp.float32),
            jax.ShapeDtypeStruct((B, 1, T), jnp.int32),
            jax.ShapeDtypeStruct((1, 1), jnp.float32),
        ],
        scratch_shapes=[pltpu.VMEM((K, C), jnp.bfloat16)],
    )(inputs, embedding, aux)

    loss = loss_sum.reshape(())
    encoding_indices = idx3.reshape(N)
    return (quant, loss, loss, encoding_indices)
